# baseline (speedup 1.0000x reference)
import functools

import jax
import jax.numpy as jnp
from jax import lax
from jax.experimental import pallas as pl
from jax.experimental.pallas import tpu as pltpu
from jax.experimental.pallas import tpu_sc as plsc

_N = 10000
_NP = 10240
_E = 320000
_D = 128

_NC = 2
_NS = 16
_NW = _NC * _NS
_CH = 128
_NRE = _E // _CH
_NG8 = _NRE // 8
_TL = _NRE - 8 * _NG8
_H0 = 32
_RPB = 48
_NB = _NP // _NS

_mesh = plsc.VectorSubcoreMesh(core_axis_name="c", subcore_axis_name="s")


@functools.partial(
    pl.kernel,
    mesh=_mesh,
    out_type=jax.ShapeDtypeStruct((2 * _NC, _N), jnp.float32),
    scratch_types=[
        pltpu.VMEM((_RPB, _CH), jnp.int32),
        pltpu.VMEM((_RPB, _CH), jnp.int32),
        pltpu.VMEM((_CH,), jnp.float32),
        pltpu.VMEM_SHARED((_N,), jnp.float32),
        pltpu.VMEM_SHARED((_N,), jnp.float32),
    ]
    + [pltpu.SemaphoreType.DMA] * 4,
)
def _deg_kernel(eidx, zeros1d, out, sidx, didx, ones_v, dego_s, degi_s,
                *dsems):
    cid = lax.axis_index("c")
    sid = lax.axis_index("s")
    wid = sid * _NC + cid
    r_lo = 8 * ((wid * _NG8) // _NW)
    r_hi = 8 * (((wid + 1) * _NG8) // _NW)

    @pl.when(sid == 0)
    def _():
        pltpu.sync_copy(zeros1d, dego_s)
        pltpu.sync_copy(zeros1d, degi_s)

    for k in range(_CH // 16):
        ones_v[pl.ds(k * 16, 16)] = jnp.ones((16,), jnp.float32)
    plsc.subcore_barrier()

    def body(g, carry):
        j0 = 2 * g
        cps = []
        for k in range(2):
            cps.append(pltpu.async_copy(ones_v,
                                        dego_s.at[sidx.at[j0 + k]],
                                        dsems[2 * k], add=True))
            cps.append(pltpu.async_copy(ones_v,
                                        degi_s.at[didx.at[j0 + k]],
                                        dsems[2 * k + 1], add=True))
        for cp in cps:
            cp.wait()
        return carry

    for half in range(2):
        r0 = pl.multiple_of(jnp.where(half == 0, r_lo, r_lo + _H0), 8)
        hn = _H0 if half == 0 else r_hi - r_lo - _H0
        pltpu.sync_copy(eidx.at[0, pl.ds(r0, _RPB)], sidx)
        pltpu.sync_copy(eidx.at[1, pl.ds(r0, _RPB)], didx)
        lax.fori_loop(0, hn // 2, body, 0)

    @pl.when(wid == 0)
    def _():
        t0 = pl.multiple_of(jnp.int32(8 * _NG8), 8)
        pltpu.sync_copy(eidx.at[0, pl.ds(t0, _TL)], sidx.at[pl.ds(0, _TL)])
        pltpu.sync_copy(eidx.at[1, pl.ds(t0, _TL)], didx.at[pl.ds(0, _TL)])
        lax.fori_loop(0, _TL // 2, body, 0)

    plsc.subcore_barrier()

    @pl.when(sid == 0)
    def _():
        pltpu.sync_copy(dego_s, out.at[2 * cid])

    @pl.when(sid == 1)
    def _():
        pltpu.sync_copy(degi_s, out.at[2 * cid + 1])


_DB = 2


@functools.partial(
    pl.kernel,
    mesh=_mesh,
    out_type=jax.ShapeDtypeStruct((_NC, _NP, _D), jnp.float32),
    scratch_types=[
        pltpu.VMEM((_RPB, _CH), jnp.int32),
        pltpu.VMEM((_RPB, _CH), jnp.int32),
        pltpu.VMEM((_DB, _CH, _D), jnp.float32),
        pltpu.VMEM_SHARED((_NP, _D), jnp.float32),
    ]
    + [pltpu.SemaphoreType.DMA] * (2 * _DB),
)
def _agg_kernel(eidx, h, zeros2d, out, sidx, didx, rows, acc, *sems):
    gs, ss = sems[:_DB], sems[_DB:]
    cid = lax.axis_index("c")
    sid = lax.axis_index("s")
    wid = sid * _NC + cid
    r_lo = 8 * ((wid * _NG8) // _NW)
    r_hi = 8 * (((wid + 1) * _NG8) // _NW)

    pltpu.sync_copy(zeros2d, acc.at[pl.ds(sid * _NB, _NB)])
    plsc.subcore_barrier()

    for half in range(2):
        r0 = pl.multiple_of(jnp.where(half == 0, r_lo, r_lo + _H0), 8)
        hn = _H0 if half == 0 else r_hi - r_lo - _H0
        pltpu.sync_copy(eidx.at[0, pl.ds(r0, _RPB)], sidx)
        pltpu.sync_copy(eidx.at[1, pl.ds(r0, _RPB)], didx)

        for b in range(_DB):
            pltpu.async_copy(h.at[sidx.at[b]], rows.at[b], gs[b])

        def group(g, carry):
            j0 = g * _DB
            for b in range(_DB):
                pltpu.make_async_copy(h.at[sidx.at[j0 + b]], rows.at[b],
                                      gs[b]).wait()
                pltpu.async_copy(rows.at[b], acc.at[didx.at[j0 + b]],
                                 ss[b], add=True)
            for b in range(_DB):
                pltpu.make_async_copy(rows.at[b], acc.at[didx.at[j0 + b]],
                                      ss[b]).wait()
                nxt = jnp.minimum(j0 + _DB + b, hn - 1)
                pltpu.async_copy(h.at[sidx.at[nxt]], rows.at[b], gs[b])
            return carry

        lax.fori_loop(0, hn // 2, group, 0)
        for b in range(_DB):
            pltpu.make_async_copy(h.at[sidx.at[hn - 1]], rows.at[b],
                                  gs[b]).wait()

    @pl.when(wid == 0)
    def _():
        t0 = pl.multiple_of(jnp.int32(8 * _NG8), 8)
        pltpu.sync_copy(eidx.at[0, pl.ds(t0, _TL)], sidx.at[pl.ds(0, _TL)])
        pltpu.sync_copy(eidx.at[1, pl.ds(t0, _TL)], didx.at[pl.ds(0, _TL)])

        def tail(j, carry):
            pltpu.async_copy(h.at[sidx.at[j]], rows.at[0], gs[0]).wait()
            pltpu.sync_copy(rows.at[0], acc.at[didx.at[j]], add=True)
            return carry

        lax.fori_loop(0, _TL, tail, 0)

    plsc.subcore_barrier()
    pltpu.sync_copy(acc.at[pl.ds(sid * _NB, _NB)],
                    out.at[cid, pl.ds(sid * _NB, _NB)])


_BN = 2000


def _mm_body(x_ref, w_ref, o_ref):
    o_ref[...] = jnp.dot(x_ref[...], w_ref[...],
                         preferred_element_type=jnp.float32)


def _scale_body(y_ref, dp_ref, o_ref):
    ns = lax.rsqrt(jnp.maximum(dp_ref[:, 0:1] + dp_ref[:, 2:3], 1.0))
    o_ref[...] = y_ref[...] * ns


def _layer_body(p_ref, dp_ref, b_ref, w_ref, o_ref):
    agg = p_ref[0] + p_ref[1]
    nd = lax.rsqrt(jnp.maximum(dp_ref[:, 1:2] + dp_ref[:, 3:4], 1.0))
    z = jnp.maximum(agg * nd + b_ref[...], 0.0)
    z = jnp.dot(z, w_ref[...], preferred_element_type=jnp.float32)
    ns = lax.rsqrt(jnp.maximum(dp_ref[:, 0:1] + dp_ref[:, 2:3], 1.0))
    o_ref[...] = z * ns


def _final_body(p_ref, dp_ref, b_ref, wo_ref, bo_ref, feat_ref, out_ref):
    agg = p_ref[0] + p_ref[1]
    nd = lax.rsqrt(jnp.maximum(dp_ref[:, 1:2] + dp_ref[:, 3:4], 1.0))
    z = jnp.maximum(agg * nd + b_ref[...], 0.0)
    nrm = jnp.sqrt(jnp.sum(z * z, axis=1, keepdims=True))
    feat = z / jnp.maximum(nrm, 1e-12)
    feat_ref[...] = feat
    out_ref[...] = jnp.dot(feat, wo_ref[...],
                           preferred_element_type=jnp.float32) + bo_ref[...]


def _tc_mm(x, W):
    return pl.pallas_call(
        _mm_body,
        grid=(_N // _BN,),
        in_specs=[
            pl.BlockSpec((_BN, _D), lambda i: (i, 0)),
            pl.BlockSpec((_D, _D), lambda i: (0, 0)),
        ],
        out_specs=pl.BlockSpec((_BN, _D), lambda i: (i, 0)),
        out_shape=jax.ShapeDtypeStruct((_N, _D), jnp.float32),
    )(x, W)


def _tc_scale(y, dparts):
    return pl.pallas_call(
        _scale_body,
        grid=(_N // _BN,),
        in_specs=[
            pl.BlockSpec((_BN, _D), lambda i: (i, 0)),
            pl.BlockSpec((_BN, 2 * _NC), lambda i: (i, 0)),
        ],
        out_specs=pl.BlockSpec((_BN, _D), lambda i: (i, 0)),
        out_shape=jax.ShapeDtypeStruct((_N, _D), jnp.float32),
    )(y, dparts)


def _tc_layer(parts, dparts, b2d, W):
    return pl.pallas_call(
        _layer_body,
        grid=(_N // _BN,),
        in_specs=[
            pl.BlockSpec((_NC, _BN, _D), lambda i: (0, i, 0)),
            pl.BlockSpec((_BN, 2 * _NC), lambda i: (i, 0)),
            pl.BlockSpec((1, _D), lambda i: (0, 0)),
            pl.BlockSpec((_D, _D), lambda i: (0, 0)),
        ],
        out_specs=pl.BlockSpec((_BN, _D), lambda i: (i, 0)),
        out_shape=jax.ShapeDtypeStruct((_N, _D), jnp.float32),
    )(parts, dparts, b2d, W)


def _tc_final(parts, dparts, b2d, Wout, bo2d):
    C = Wout.shape[1]
    return pl.pallas_call(
        _final_body,
        grid=(_N // _BN,),
        in_specs=[
            pl.BlockSpec((_NC, _BN, _D), lambda i: (0, i, 0)),
            pl.BlockSpec((_BN, 2 * _NC), lambda i: (i, 0)),
            pl.BlockSpec((1, _D), lambda i: (0, 0)),
            pl.BlockSpec((_D, C), lambda i: (0, 0)),
            pl.BlockSpec((1, C), lambda i: (0, 0)),
        ],
        out_specs=[
            pl.BlockSpec((_BN, _D), lambda i: (i, 0)),
            pl.BlockSpec((_BN, C), lambda i: (i, 0)),
        ],
        out_shape=[
            jax.ShapeDtypeStruct((_N, _D), jnp.float32),
            jax.ShapeDtypeStruct((_N, C), jnp.float32),
        ],
    )(parts, dparts, b2d, Wout, bo2d)


def kernel(x, edge_index, W1, b1, W2, b2, Wout, bout):
    eidx = edge_index.reshape(2, _NRE, _CH)
    zeros1d = jnp.zeros((_N,), jnp.float32)
    zeros2d = jnp.zeros((_NB, _D), jnp.float32)

    y1 = _tc_mm(x, W1)
    dparts = _deg_kernel(eidx, zeros1d).T
    h0 = _tc_scale(y1, dparts)
    p1 = _agg_kernel(eidx, h0, zeros2d)
    h1s = _tc_layer(p1, dparts, b1.reshape(1, _D), W2)
    p2 = _agg_kernel(eidx, h1s, zeros2d)
    feat, out = _tc_final(p2, dparts, b2.reshape(1, _D), Wout,
                          bout.reshape(1, -1))
    return (out, feat)


# speedup vs baseline: 1.2512x; 1.2512x over previous
import functools

import jax
import jax.numpy as jnp
from jax import lax
from jax.experimental import pallas as pl
from jax.experimental.pallas import tpu as pltpu
from jax.experimental.pallas import tpu_sc as plsc

_N = 10000
_NP = 10240
_E = 320000
_D = 128

_NC = 2
_NS = 16
_NW = _NC * _NS
_CH = 128
_NRE = _E // _CH
_NG8 = _NRE // 8
_TL = _NRE - 8 * _NG8
_H0 = 32
_RPB = 48
_NB = _NP // _NS

_mesh = plsc.VectorSubcoreMesh(core_axis_name="c", subcore_axis_name="s")


@functools.partial(
    pl.kernel,
    mesh=_mesh,
    out_type=jax.ShapeDtypeStruct((2 * _NC, _N), jnp.float32),
    scratch_types=[
        pltpu.VMEM((_RPB, _CH), jnp.int32),
        pltpu.VMEM((_RPB, _CH), jnp.int32),
        pltpu.VMEM((_CH,), jnp.float32),
        pltpu.VMEM_SHARED((_N,), jnp.float32),
        pltpu.VMEM_SHARED((_N,), jnp.float32),
    ]
    + [pltpu.SemaphoreType.DMA] * 8,
)
def _deg_kernel(eidx, zeros1d, out, sidx, didx, ones_v, dego_s, degi_s,
                *dsems):
    cid = lax.axis_index("c")
    sid = lax.axis_index("s")
    wid = sid * _NC + cid
    r_lo = 8 * ((wid * _NG8) // _NW)
    r_hi = 8 * (((wid + 1) * _NG8) // _NW)

    @pl.when(sid == 0)
    def _():
        pltpu.sync_copy(zeros1d, dego_s)
        pltpu.sync_copy(zeros1d, degi_s)

    for k in range(_CH // 16):
        ones_v[pl.ds(k * 16, 16)] = jnp.ones((16,), jnp.float32)
    plsc.subcore_barrier()

    def body(g, carry):
        j0 = 4 * g
        cps = []
        for k in range(4):
            cps.append(pltpu.async_copy(ones_v,
                                        dego_s.at[sidx.at[j0 + k]],
                                        dsems[2 * k], add=True))
            cps.append(pltpu.async_copy(ones_v,
                                        degi_s.at[didx.at[j0 + k]],
                                        dsems[2 * k + 1], add=True))
        for cp in cps:
            cp.wait()
        return carry

    for half in range(2):
        r0 = pl.multiple_of(jnp.where(half == 0, r_lo, r_lo + _H0), 8)
        hn = _H0 if half == 0 else r_hi - r_lo - _H0
        pltpu.sync_copy(eidx.at[0, pl.ds(r0, _RPB)], sidx)
        pltpu.sync_copy(eidx.at[1, pl.ds(r0, _RPB)], didx)
        lax.fori_loop(0, hn // 4, body, 0)

    @pl.when(wid == 0)
    def _():
        t0 = pl.multiple_of(jnp.int32(8 * _NG8), 8)
        pltpu.sync_copy(eidx.at[0, pl.ds(t0, _TL)], sidx.at[pl.ds(0, _TL)])
        pltpu.sync_copy(eidx.at[1, pl.ds(t0, _TL)], didx.at[pl.ds(0, _TL)])
        lax.fori_loop(0, _TL // 4, body, 0)

    plsc.subcore_barrier()

    @pl.when(sid == 0)
    def _():
        pltpu.sync_copy(dego_s, out.at[2 * cid])

    @pl.when(sid == 1)
    def _():
        pltpu.sync_copy(degi_s, out.at[2 * cid + 1])


_DB = 2


@functools.partial(
    pl.kernel,
    mesh=_mesh,
    out_type=jax.ShapeDtypeStruct((_NC, _NP, _D), jnp.float32),
    scratch_types=[
        pltpu.VMEM((_RPB, _CH), jnp.int32),
        pltpu.VMEM((_RPB, _CH), jnp.int32),
        pltpu.VMEM((_DB, _CH, _D), jnp.float32),
        pltpu.VMEM_SHARED((_NP, _D), jnp.float32),
    ]
    + [pltpu.SemaphoreType.DMA] * _DB,
)
def _agg_kernel(eidx, h, out, sidx, didx, rows, acc, *gs):
    cid = lax.axis_index("c")
    sid = lax.axis_index("s")
    wid = sid * _NC + cid
    r_lo = 8 * ((wid * _NG8) // _NW)
    r_hi = 8 * (((wid + 1) * _NG8) // _NW)

    zv = jnp.zeros((16,), jnp.float32)

    def zfill(g, carry):
        rows[0, g // 8, pl.ds((g % 8) * 16, 16)] = zv
        return carry

    lax.fori_loop(0, _CH * _D // 16, zfill, 0)
    for q in range(_NB // _CH):
        pltpu.sync_copy(rows.at[0], acc.at[pl.ds(sid * _NB + q * _CH, _CH)])
    plsc.subcore_barrier()

    for half in range(2):
        r0 = pl.multiple_of(jnp.where(half == 0, r_lo, r_lo + _H0), 8)
        hn = _H0 if half == 0 else r_hi - r_lo - _H0
        pltpu.sync_copy(eidx.at[0, pl.ds(r0, _RPB)], sidx)
        pltpu.sync_copy(eidx.at[1, pl.ds(r0, _RPB)], didx)

        for b in range(_DB):
            pltpu.async_copy(h.at[sidx.at[b]], rows.at[b], gs[b])

        def group(g, carry):
            j0 = g * _DB
            for b in range(_DB):
                pltpu.make_async_copy(h.at[sidx.at[j0 + b]], rows.at[b],
                                      gs[b]).wait()
                pltpu.sync_copy(rows.at[b], acc.at[didx.at[j0 + b]],
                                add=True)
                nxt = jnp.minimum(j0 + _DB + b, hn - 1)
                pltpu.async_copy(h.at[sidx.at[nxt]], rows.at[b], gs[b])
            return carry

        lax.fori_loop(0, hn // 2, group, 0)
        for b in range(_DB):
            pltpu.make_async_copy(h.at[sidx.at[hn - 1]], rows.at[b],
                                  gs[b]).wait()

    @pl.when(wid == 0)
    def _():
        t0 = pl.multiple_of(jnp.int32(8 * _NG8), 8)
        pltpu.sync_copy(eidx.at[0, pl.ds(t0, _TL)], sidx.at[pl.ds(0, _TL)])
        pltpu.sync_copy(eidx.at[1, pl.ds(t0, _TL)], didx.at[pl.ds(0, _TL)])

        def tail(j, carry):
            pltpu.async_copy(h.at[sidx.at[j]], rows.at[0], gs[0]).wait()
            pltpu.sync_copy(rows.at[0], acc.at[didx.at[j]], add=True)
            return carry

        lax.fori_loop(0, _TL, tail, 0)

    plsc.subcore_barrier()
    pltpu.sync_copy(acc.at[pl.ds(sid * _NB, _NB)],
                    out.at[cid, pl.ds(sid * _NB, _NB)])


_BN = 2000


def _mm_body(x_ref, w_ref, o_ref):
    o_ref[...] = jnp.dot(x_ref[...], w_ref[...],
                         preferred_element_type=jnp.float32)


def _scale_body(y_ref, dp_ref, o_ref):
    ns = lax.rsqrt(jnp.maximum(dp_ref[:, 0:1] + dp_ref[:, 2:3], 1.0))
    o_ref[...] = y_ref[...] * ns


def _layer_body(p_ref, dp_ref, b_ref, w_ref, o_ref):
    agg = p_ref[0] + p_ref[1]
    nd = lax.rsqrt(jnp.maximum(dp_ref[:, 1:2] + dp_ref[:, 3:4], 1.0))
    z = jnp.maximum(agg * nd + b_ref[...], 0.0)
    z = jnp.dot(z, w_ref[...], preferred_element_type=jnp.float32)
    ns = lax.rsqrt(jnp.maximum(dp_ref[:, 0:1] + dp_ref[:, 2:3], 1.0))
    o_ref[...] = z * ns


def _final_body(p_ref, dp_ref, b_ref, wo_ref, bo_ref, feat_ref, out_ref):
    agg = p_ref[0] + p_ref[1]
    nd = lax.rsqrt(jnp.maximum(dp_ref[:, 1:2] + dp_ref[:, 3:4], 1.0))
    z = jnp.maximum(agg * nd + b_ref[...], 0.0)
    nrm = jnp.sqrt(jnp.sum(z * z, axis=1, keepdims=True))
    feat = z / jnp.maximum(nrm, 1e-12)
    feat_ref[...] = feat
    out_ref[...] = jnp.dot(feat, wo_ref[...],
                           preferred_element_type=jnp.float32) + bo_ref[...]


def _tc_mm(x, W):
    return pl.pallas_call(
        _mm_body,
        grid=(_N // _BN,),
        in_specs=[
            pl.BlockSpec((_BN, _D), lambda i: (i, 0)),
            pl.BlockSpec((_D, _D), lambda i: (0, 0)),
        ],
        out_specs=pl.BlockSpec((_BN, _D), lambda i: (i, 0)),
        out_shape=jax.ShapeDtypeStruct((_N, _D), jnp.float32),
    )(x, W)


def _tc_scale(y, dparts):
    return pl.pallas_call(
        _scale_body,
        grid=(_N // _BN,),
        in_specs=[
            pl.BlockSpec((_BN, _D), lambda i: (i, 0)),
            pl.BlockSpec((_BN, 2 * _NC), lambda i: (i, 0)),
        ],
        out_specs=pl.BlockSpec((_BN, _D), lambda i: (i, 0)),
        out_shape=jax.ShapeDtypeStruct((_N, _D), jnp.float32),
    )(y, dparts)


def _tc_layer(parts, dparts, b2d, W):
    return pl.pallas_call(
        _layer_body,
        grid=(_N // _BN,),
        in_specs=[
            pl.BlockSpec((_NC, _BN, _D), lambda i: (0, i, 0)),
            pl.BlockSpec((_BN, 2 * _NC), lambda i: (i, 0)),
            pl.BlockSpec((1, _D), lambda i: (0, 0)),
            pl.BlockSpec((_D, _D), lambda i: (0, 0)),
        ],
        out_specs=pl.BlockSpec((_BN, _D), lambda i: (i, 0)),
        out_shape=jax.ShapeDtypeStruct((_N, _D), jnp.float32),
    )(parts, dparts, b2d, W)


def _tc_final(parts, dparts, b2d, Wout, bo2d):
    C = Wout.shape[1]
    return pl.pallas_call(
        _final_body,
        grid=(_N // _BN,),
        in_specs=[
            pl.BlockSpec((_NC, _BN, _D), lambda i: (0, i, 0)),
            pl.BlockSpec((_BN, 2 * _NC), lambda i: (i, 0)),
            pl.BlockSpec((1, _D), lambda i: (0, 0)),
            pl.BlockSpec((_D, C), lambda i: (0, 0)),
            pl.BlockSpec((1, C), lambda i: (0, 0)),
        ],
        out_specs=[
            pl.BlockSpec((_BN, _D), lambda i: (i, 0)),
            pl.BlockSpec((_BN, C), lambda i: (i, 0)),
        ],
        out_shape=[
            jax.ShapeDtypeStruct((_N, _D), jnp.float32),
            jax.ShapeDtypeStruct((_N, C), jnp.float32),
        ],
    )(parts, dparts, b2d, Wout, bo2d)


def kernel(x, edge_index, W1, b1, W2, b2, Wout, bout):
    eidx = edge_index.reshape(2, _NRE, _CH)
    zeros1d = jnp.zeros((_N,), jnp.float32)

    y1 = _tc_mm(x, W1)
    dparts = _deg_kernel(eidx, zeros1d).T
    h0 = _tc_scale(y1, dparts)
    p1 = _agg_kernel(eidx, h0)
    h1s = _tc_layer(p1, dparts, b1.reshape(1, _D), W2)
    p2 = _agg_kernel(eidx, h1s)
    feat, out = _tc_final(p2, dparts, b2.reshape(1, _D), Wout,
                          bout.reshape(1, -1))
    return (out, feat)
